# baseline (speedup 1.0000x reference)
import jax
import jax.numpy as jnp
from jax import lax
from jax.experimental import pallas as pl
from jax.experimental.pallas import tpu as pltpu

_K = 16
_ROWTILE = 8


def _topk_body(x_ref, y_ref, d_ref, i_ref):
    n = x_ref.shape[2]
    c_dim = x_ref.shape[3]
    m = y_ref.shape[3]
    lane_iota = lax.broadcasted_iota(jnp.int32, (_ROWTILE, m), 1)

    def tile_body(nt, carry):
        base = nt * _ROWTILE
        x_tile = x_ref[0, 0, pl.ds(base, _ROWTILE), :]
        acc = jnp.zeros((_ROWTILE, m), jnp.float32)
        for c0 in range(0, c_dim, 4):
            chain = None
            for c in range(c0, min(c0 + 4, c_dim)):
                xa = x_tile[:, c:c + 1]
                yb = jnp.reshape(y_ref[0, 0, c, :], (1, m))
                d = xa - yb
                s = d * d
                chain = s if chain is None else chain + s
            acc = acc + chain
        vals = jnp.where(acc == 0.0, 0.0, acc * lax.rsqrt(acc))

        dists = []
        idxs = []
        for _ in range(_K):
            rowmin = jnp.min(vals, axis=1, keepdims=True)
            eq = vals == rowmin
            rowidx = jnp.min(jnp.where(eq, lane_iota, m), axis=1,
                             keepdims=True)
            dists.append(rowmin)
            idxs.append(rowidx)
            vals = jnp.where(lane_iota == rowidx, jnp.inf, vals)
        d_ref[0, 0, pl.ds(base, _ROWTILE), :] = jnp.concatenate(dists, axis=1)
        i_ref[0, 0, pl.ds(base, _ROWTILE), :] = jnp.concatenate(idxs, axis=1)
        return carry

    lax.fori_loop(0, n // _ROWTILE, tile_body, 0)


def kernel(a, b, k):
    del k
    bsz, c_dim, n = a.shape
    m = b.shape[2]

    xs = jnp.stack([jnp.swapaxes(a, 1, 2), jnp.swapaxes(b, 1, 2)])
    ys = jnp.stack([b, a])

    dstack, istack = pl.pallas_call(
        _topk_body,
        grid=(2, bsz),
        in_specs=[
            pl.BlockSpec((1, 1, n, c_dim), lambda d, bb: (d, bb, 0, 0)),
            pl.BlockSpec((1, 1, c_dim, m), lambda d, bb: (d, bb, 0, 0)),
        ],
        out_specs=[
            pl.BlockSpec((1, 1, n, _K), lambda d, bb: (d, bb, 0, 0)),
            pl.BlockSpec((1, 1, n, _K), lambda d, bb: (d, bb, 0, 0)),
        ],
        out_shape=[
            jax.ShapeDtypeStruct((2, bsz, n, _K), jnp.float32),
            jax.ShapeDtypeStruct((2, bsz, n, _K), jnp.int32),
        ],
    )(xs, ys)

    dist1p = jnp.swapaxes(dstack[0], 1, 2)
    dist2 = jnp.swapaxes(dstack[1], 1, 2)
    idx1p = jnp.swapaxes(istack[0], 1, 2).astype(jnp.int64)
    idx2 = jnp.swapaxes(istack[1], 1, 2).astype(jnp.int64)
    return (dist1p, dist2, idx1p, idx2)


# speedup vs baseline: 4.3273x; 4.3273x over previous
import jax
import jax.numpy as jnp
from jax import lax
from jax.experimental import pallas as pl
from jax.experimental.pallas import tpu as pltpu

_K = 16
_DTILE = 16
_TTILE = 8
_LANES = 128


def _body(x_ref, y_ref, d_ref, i_ref, s_ref):
    mm = x_ref.shape[2]
    c_dim = x_ref.shape[3]
    n = y_ref.shape[3]

    def dist_tile(mt, carry):
        base = mt * _DTILE
        acc = jnp.zeros((_DTILE, n), jnp.float32)
        for c0 in range(0, c_dim, 4):
            chain = None
            for c in range(c0, min(c0 + 4, c_dim)):
                xa = x_ref[0, 0, pl.ds(base, _DTILE), c:c + 1]
                yb = jnp.reshape(y_ref[0, 0, c, :], (1, n))
                d = xa - yb
                s = d * d
                chain = s if chain is None else chain + s
            acc = acc + chain
        s_ref[pl.ds(base, _DTILE), :] = jnp.where(
            acc == 0.0, 0.0, acc * lax.rsqrt(acc))
        return carry

    lax.fori_loop(0, mm // _DTILE, dist_tile, 0)

    sub_iota = lax.broadcasted_iota(jnp.int32, (_TTILE, _LANES), 0)
    nsteps = mm // _TTILE
    big = jnp.int32(1 << 30)
    for lb in range(n // _LANES):
        lanes = pl.ds(lb * _LANES, _LANES)
        idx_prev = jnp.full((1, _LANES), -1, jnp.int32)
        for j in range(_K):
            if j == 0:
                def min_pass(t, runmin):
                    v = s_ref[pl.ds(t * _TTILE, _TTILE), lanes]
                    return jnp.minimum(runmin, v)
            else:
                def min_pass(t, runmin, ip=idx_prev):
                    row = pl.ds(t * _TTILE, _TTILE)
                    v = s_ref[row, lanes]
                    veff = jnp.where(sub_iota + t * _TTILE == ip,
                                     jnp.inf, v)
                    s_ref[row, lanes] = veff
                    return jnp.minimum(runmin, veff)
            runmin = lax.fori_loop(
                0, nsteps, min_pass,
                jnp.full((_TTILE, _LANES), jnp.inf, jnp.float32))
            colmin = jnp.min(runmin, axis=0, keepdims=True)

            def idx_pass(t, runidx, cm=colmin):
                v = s_ref[pl.ds(t * _TTILE, _TTILE), lanes]
                cand = jnp.where(v == cm, sub_iota + t * _TTILE, big)
                return jnp.minimum(runidx, cand)
            runidx = lax.fori_loop(
                0, nsteps, idx_pass,
                jnp.full((_TTILE, _LANES), big, jnp.int32))
            colidx = jnp.min(runidx, axis=0, keepdims=True)

            d_ref[0, 0, j, lanes] = jnp.reshape(colmin, (_LANES,))
            i_ref[0, 0, j, lanes] = jnp.reshape(colidx, (_LANES,))
            idx_prev = colidx


def kernel(a, b, k):
    del k
    bsz, c_dim, n = a.shape
    m = b.shape[2]

    xs = jnp.stack([jnp.swapaxes(b, 1, 2), jnp.swapaxes(a, 1, 2)])
    ys = jnp.stack([a, b])

    dstack, istack = pl.pallas_call(
        _body,
        grid=(2, bsz),
        in_specs=[
            pl.BlockSpec((1, 1, m, c_dim), lambda d, bb: (d, bb, 0, 0)),
            pl.BlockSpec((1, 1, c_dim, n), lambda d, bb: (d, bb, 0, 0)),
        ],
        out_specs=[
            pl.BlockSpec((1, 1, _K, n), lambda d, bb: (d, bb, 0, 0)),
            pl.BlockSpec((1, 1, _K, n), lambda d, bb: (d, bb, 0, 0)),
        ],
        out_shape=[
            jax.ShapeDtypeStruct((2, bsz, _K, n), jnp.float32),
            jax.ShapeDtypeStruct((2, bsz, _K, n), jnp.int32),
        ],
        scratch_shapes=[pltpu.VMEM((m, n), jnp.float32)],
    )(xs, ys)

    dist1p = dstack[0]
    dist2 = dstack[1]
    idx1p = istack[0].astype(jnp.int64)
    idx2 = istack[1].astype(jnp.int64)
    return (dist1p, dist2, idx1p, idx2)


# speedup vs baseline: 15.2903x; 3.5335x over previous
import jax
import jax.numpy as jnp
from jax import lax
from jax.experimental import pallas as pl
from jax.experimental.pallas import tpu as pltpu

_K = 16
_DTILE = 16
_TTILE = 8
_LANES = 128


def _body(x_ref, y_ref, d_ref, i_ref, s_ref):
    mm = x_ref.shape[2]
    c_dim = x_ref.shape[3]
    n = y_ref.shape[3]

    def dist_tile(mt, carry):
        base = mt * _DTILE
        acc = jnp.zeros((_DTILE, n), jnp.float32)
        for c0 in range(0, c_dim, 4):
            chain = None
            for c in range(c0, min(c0 + 4, c_dim)):
                xa = x_ref[0, 0, pl.ds(base, _DTILE), c:c + 1]
                yb = jnp.reshape(y_ref[0, 0, c, :], (1, n))
                d = xa - yb
                s = d * d
                chain = s if chain is None else chain + s
            acc = acc + chain
        s_ref[pl.ds(base, _DTILE), :] = jnp.where(
            acc == 0.0, 0.0, acc * lax.rsqrt(acc))
        return carry

    lax.fori_loop(0, mm // _DTILE, dist_tile, 0)

    chunk = 128
    nchunks = mm // chunk
    sub_iota = lax.broadcasted_iota(jnp.int32, (chunk, _LANES), 0)
    big = jnp.int32(1 << 30)
    for lb in range(n // _LANES):
        lanes = pl.ds(lb * _LANES, _LANES)
        idx_prev = None
        for j in range(_K):
            partmins = []
            for t in range(nchunks):
                row = pl.ds(t * chunk, chunk)
                v = s_ref[row, lanes]
                if idx_prev is not None:
                    v = jnp.where(sub_iota + t * chunk == idx_prev,
                                  jnp.inf, v)
                    s_ref[row, lanes] = v
                partmins.append(jnp.min(v, axis=0, keepdims=True))
            while len(partmins) > 1:
                partmins = [jnp.minimum(*partmins[i:i + 2])
                            if i + 1 < len(partmins) else partmins[i]
                            for i in range(0, len(partmins), 2)]
            colmin = partmins[0]
            partidx = []
            for t in range(nchunks):
                v = s_ref[pl.ds(t * chunk, chunk), lanes]
                cand = jnp.where(v == colmin, sub_iota + t * chunk, big)
                partidx.append(jnp.min(cand, axis=0, keepdims=True))
            while len(partidx) > 1:
                partidx = [jnp.minimum(*partidx[i:i + 2])
                           if i + 1 < len(partidx) else partidx[i]
                           for i in range(0, len(partidx), 2)]
            colidx = partidx[0]

            d_ref[0, 0, j, lanes] = jnp.reshape(colmin, (_LANES,))
            i_ref[0, 0, j, lanes] = jnp.reshape(colidx, (_LANES,))
            idx_prev = colidx


def kernel(a, b, k):
    del k
    bsz, c_dim, n = a.shape
    m = b.shape[2]

    xs = jnp.stack([jnp.swapaxes(b, 1, 2), jnp.swapaxes(a, 1, 2)])
    ys = jnp.stack([a, b])

    dstack, istack = pl.pallas_call(
        _body,
        grid=(2, bsz),
        in_specs=[
            pl.BlockSpec((1, 1, m, c_dim), lambda d, bb: (d, bb, 0, 0)),
            pl.BlockSpec((1, 1, c_dim, n), lambda d, bb: (d, bb, 0, 0)),
        ],
        out_specs=[
            pl.BlockSpec((1, 1, _K, n), lambda d, bb: (d, bb, 0, 0)),
            pl.BlockSpec((1, 1, _K, n), lambda d, bb: (d, bb, 0, 0)),
        ],
        out_shape=[
            jax.ShapeDtypeStruct((2, bsz, _K, n), jnp.float32),
            jax.ShapeDtypeStruct((2, bsz, _K, n), jnp.int32),
        ],
        scratch_shapes=[pltpu.VMEM((m, n), jnp.float32)],
    )(xs, ys)

    dist1p = dstack[0]
    dist2 = dstack[1]
    idx1p = istack[0].astype(jnp.int64)
    idx2 = istack[1].astype(jnp.int64)
    return (dist1p, dist2, idx1p, idx2)
